# baseline (speedup 1.0000x reference)
import functools

import jax
import jax.numpy as jnp
from jax import lax
from jax.experimental import pallas as pl
from jax.experimental.pallas import tpu as pltpu
from jax.experimental.pallas import tpu_sc as plsc

_NUM_CORES = 2
_NUM_SUBCORES = 16
_NUM_WORKERS = _NUM_CORES * _NUM_SUBCORES
_LANES = 16


def kernel(x, table):
    B, S = x.shape
    V, D = table.shape
    BC = B // 128
    DG = D // 8
    assert BC == _NUM_WORKERS and D % 8 == 0

    xT3 = x.T.reshape(S, BC, 128)
    mesh = plsc.VectorSubcoreMesh(core_axis_name="c", subcore_axis_name="s")

    @functools.partial(
        pl.kernel,
        out_type=jax.ShapeDtypeStruct((S, DG, BC, 1024), jnp.float32),
        mesh=mesh,
        scratch_types=[
            pltpu.VMEM((S, 128), jnp.int32),
            pltpu.VMEM((2, 128, D), jnp.float32),
            pltpu.VMEM((2, DG * 1024), jnp.float32),
            pltpu.SemaphoreType.DMA((2,)),
            pltpu.SemaphoreType.DMA((2,)),
        ],
        compiler_params=pltpu.CompilerParams(use_tc_tiling_on_sc=False, needs_layout_passes=False),
    )
    def emb(idx_hbm, table_hbm, out_hbm, idx_v, vin, tbuf, gsem, osem):
        wid = lax.axis_index("s") * _NUM_CORES + lax.axis_index("c")
        pltpu.sync_copy(idx_hbm.at[:, wid], idx_v)

        base = [lax.iota(jnp.int32, _LANES) * 128 + c * 2048 for c in range(4)]

        def fire_gather(s, sl):
            pltpu.async_copy(table_hbm.at[idx_v.at[s]], vin.at[sl], gsem.at[sl])

        def drain_gather(sl):
            pltpu.make_async_copy(
                table_hbm.at[pl.ds(0, 128)], vin.at[sl], gsem.at[sl]).wait()

        def drain_out(sl):
            for dg in range(DG):
                pltpu.make_async_copy(
                    tbuf.at[sl, pl.ds(dg * 1024, 1024)],
                    out_hbm.at[0, dg, wid], osem.at[sl]).wait()

        fire_gather(0, 0)

        @pl.loop(0, S, step=2)
        def _grp(s0):
            for sl in range(2):
                s = s0 + sl
                nsl = (sl + 1) % 2

                @pl.when(s + 1 < S)
                def _fire_next():
                    fire_gather(s + 1, nsl)

                drain_gather(sl)

                @pl.when(s >= 2)
                def _drain_prev():
                    drain_out(sl)

                dst = tbuf.at[sl]
                for l in range(128):
                    for c in range(4):
                        vals = vin[sl, l, pl.ds(c * _LANES, _LANES)]
                        plsc.store_scatter(dst, [base[c] + l], vals)

                for dg in range(DG):
                    pltpu.async_copy(
                        tbuf.at[sl, pl.ds(dg * 1024, 1024)],
                        out_hbm.at[s, dg, wid], osem.at[sl])

        for sl in range(2):
            drain_out(sl)

    o4 = emb(xT3, table)
    o5 = o4.reshape(S, DG, BC, 8, 128)
    return o5.transpose(2, 4, 0, 1, 3).reshape(B, S, D)


# speedup vs baseline: 1.2347x; 1.2347x over previous
import functools

import jax
import jax.numpy as jnp
from jax import lax
from jax.experimental import pallas as pl
from jax.experimental.pallas import tpu as pltpu
from jax.experimental.pallas import tpu_sc as plsc

_NUM_CORES = 2
_NUM_SUBCORES = 16
_NUM_WORKERS = _NUM_CORES * _NUM_SUBCORES
_LANES = 16


def kernel(x, table):
    B, S = x.shape
    V, D = table.shape
    BC = B // 128
    DG = D // 8
    assert BC == _NUM_WORKERS and D % 8 == 0

    xT3 = x.T.reshape(S, BC, 128)
    mesh = plsc.VectorSubcoreMesh(core_axis_name="c", subcore_axis_name="s")

    @functools.partial(
        pl.kernel,
        out_type=jax.ShapeDtypeStruct((S, DG, BC, 1024), jnp.float32),
        mesh=mesh,
        scratch_types=[
            pltpu.VMEM((S, 128), jnp.int32),
            pltpu.VMEM((2, 128, D), jnp.float32),
            pltpu.VMEM((2, DG * 1024), jnp.float32),
            pltpu.SemaphoreType.DMA((2,)),
            pltpu.SemaphoreType.DMA((2,)),
        ],
        compiler_params=pltpu.CompilerParams(use_tc_tiling_on_sc=False, needs_layout_passes=False),
    )
    def emb(idx_hbm, table_hbm, out_hbm, idx_v, vin, tbuf, gsem, osem):
        wid = lax.axis_index("s") * _NUM_CORES + lax.axis_index("c")
        pltpu.sync_copy(idx_hbm.at[:, wid], idx_v)

        base = [lax.iota(jnp.int32, _LANES) * 128 + c * 2048 for c in range(4)]

        def fire_gather(s, sl):
            pltpu.async_copy(table_hbm.at[idx_v.at[s]], vin.at[sl], gsem.at[sl])

        def drain_gather(sl):
            pltpu.make_async_copy(
                table_hbm.at[pl.ds(0, 128)], vin.at[sl], gsem.at[sl]).wait()

        def drain_out(sl):
            for dg in range(DG):
                pltpu.make_async_copy(
                    tbuf.at[sl, pl.ds(dg * 1024, 1024)],
                    out_hbm.at[0, dg, wid], osem.at[sl]).wait()

        fire_gather(0, 0)

        @pl.loop(0, S, step=2)
        def _grp(s0):
            for sl in range(2):
                s = s0 + sl
                nsl = (sl + 1) % 2

                @pl.when(s + 1 < S)
                def _fire_next():
                    fire_gather(s + 1, nsl)

                drain_gather(sl)

                @pl.when(s >= 2)
                def _drain_prev():
                    drain_out(sl)

                dst = tbuf.at[sl]
                src = vin.at[sl]

                @plsc.parallel_loop(0, 128, step=1, unroll=8)
                def _transpose(l):
                    for c in range(4):
                        vals = src[l, pl.ds(c * _LANES, _LANES)]
                        plsc.store_scatter(dst, [base[c] + l], vals)

                for dg in range(DG):
                    pltpu.async_copy(
                        tbuf.at[sl, pl.ds(dg * 1024, 1024)],
                        out_hbm.at[s, dg, wid], osem.at[sl])

        for sl in range(2):
            drain_out(sl)

    o4 = emb(xT3, table)
    o5 = o4.reshape(S, DG, BC, 8, 128)
    return o5.transpose(2, 4, 0, 1, 3).reshape(B, S, D)


# speedup vs baseline: 2.0472x; 1.6581x over previous
import functools

import jax
import jax.numpy as jnp
from jax import lax
from jax.experimental import pallas as pl
from jax.experimental.pallas import tpu as pltpu
from jax.experimental.pallas import tpu_sc as plsc

_NUM_CORES = 2
_NUM_SUBCORES = 16
_NUM_WORKERS = _NUM_CORES * _NUM_SUBCORES
_LANES = 16


def kernel(x, table):
    B, S = x.shape
    V, D = table.shape
    BC = B // 128
    DG = D // 8
    assert BC == _NUM_WORKERS and D % 8 == 0

    xT3 = x.T.reshape(S, BC, 128)
    mesh = plsc.VectorSubcoreMesh(core_axis_name="c", subcore_axis_name="s")

    @functools.partial(
        pl.kernel,
        out_type=jax.ShapeDtypeStruct((S, DG, BC, 8, 128), jnp.float32),
        mesh=mesh,
        scratch_types=[
            pltpu.VMEM((S, 128), jnp.int32),
            pltpu.VMEM((2, 128, D), jnp.float32),
            pltpu.VMEM((2, D, 129), jnp.float32),
            pltpu.SemaphoreType.DMA((2,)),
            pltpu.SemaphoreType.DMA((2,)),
        ],
        compiler_params=pltpu.CompilerParams(use_tc_tiling_on_sc=False, needs_layout_passes=False),
    )
    def emb(idx_hbm, table_hbm, out_hbm, idx_v, vin, tbuf, gsem, osem):
        wid = lax.axis_index("s") * _NUM_CORES + lax.axis_index("c")
        pltpu.sync_copy(idx_hbm.at[:, wid], idx_v)

        base = [lax.iota(jnp.int32, _LANES) + c * _LANES for c in range(4)]

        def fire_gather(s, sl):
            pltpu.async_copy(table_hbm.at[idx_v.at[s]], vin.at[sl], gsem.at[sl])

        def drain_gather(sl):
            pltpu.make_async_copy(
                table_hbm.at[pl.ds(0, 128)], vin.at[sl], gsem.at[sl]).wait()

        def drain_out(sl):
            for dg in range(DG):
                pltpu.make_async_copy(
                    tbuf.at[sl, pl.ds(dg * 8, 8), pl.ds(0, 128)],
                    out_hbm.at[0, dg, wid], osem.at[sl]).wait()

        fire_gather(0, 0)

        @pl.loop(0, S, step=2)
        def _grp(s0):
            for sl in range(2):
                s = s0 + sl
                nsl = (sl + 1) % 2

                @pl.when(s + 1 < S)
                def _fire_next():
                    fire_gather(s + 1, nsl)

                drain_gather(sl)

                @pl.when(s >= 2)
                def _drain_prev():
                    drain_out(sl)

                dst = tbuf.at[sl]
                src = vin.at[sl]

                @plsc.parallel_loop(0, 128, step=1, unroll=8)
                def _transpose(l):
                    lvec = jnp.full((_LANES,), l, jnp.int32)
                    for c in range(4):
                        vals = src[l, pl.ds(c * _LANES, _LANES)]
                        plsc.store_scatter(dst, [base[c], lvec], vals)

                for dg in range(DG):
                    pltpu.async_copy(
                        tbuf.at[sl, pl.ds(dg * 8, 8), pl.ds(0, 128)],
                        out_hbm.at[s, dg, wid], osem.at[sl])

        for sl in range(2):
            drain_out(sl)

    o5 = emb(xT3, table)
    return o5.transpose(2, 4, 0, 1, 3).reshape(B, S, D)
